# baseline (speedup 1.0000x reference)
import functools

import jax
import jax.numpy as jnp
from jax import lax
from jax.experimental import pallas as pl
from jax.experimental.pallas import tpu as pltpu
from jax.experimental.pallas import tpu_sc as plsc


def kernel(x):
    B, S, C = x.shape
    h = int(round(S ** 0.5))
    hi, hj = h - 1, h - 2
    K = hi * hj

    info = plsc.get_sparse_core_info()
    nw = info.num_cores * info.num_subcores
    per_w = B // nw

    GR = 4
    chunks = []
    i0 = 0
    while i0 < hi:
        chunks.append((i0, min(GR, hi - i0)))
        i0 += GR
    items = [(t, c0, ngr) for t in range(per_w) for (c0, ngr) in chunks]
    n_items = len(items)
    NC16 = C // 16
    TAILN = (hi % GR) * hj
    TAILA = TAILN - (TAILN % 8)

    mesh = plsc.VectorSubcoreMesh(core_axis_name="c", subcore_axis_name="s")

    @functools.partial(
        pl.kernel,
        mesh=mesh,
        out_type=jax.ShapeDtypeStruct((B, K, C), jnp.float32),
        scratch_types=[
            pltpu.VMEM((GR * h, C), jnp.float32),
            pltpu.VMEM((GR * h, C), jnp.float32),
            pltpu.VMEM((GR * hj, C), jnp.float32),
            pltpu.VMEM((GR * hj, C), jnp.float32),
            pltpu.VMEM((TAILN - TAILA, C), jnp.float32),
            pltpu.SemaphoreType.DMA,
            pltpu.SemaphoreType.DMA,
            pltpu.SemaphoreType.DMA,
            pltpu.SemaphoreType.DMA,
            pltpu.SemaphoreType.DMA,
        ],
        compiler_params=pltpu.CompilerParams(
            needs_layout_passes=False,
            disable_bounds_checks=True,
            disable_semaphore_checks=True,
        ),
    )
    def copy_k(x_hbm, out_hbm, ib0, ib1, ob0, ob1, tb,
               is0, is1, os0, os1, tsem):
        wid = lax.axis_index("s") * info.num_cores + lax.axis_index("c")
        ibufs, obufs = (ib0, ib1), (ob0, ob1)
        isems, osems = (is0, is1), (os0, os1)

        def is_tail(k):
            return items[k][2] != GR

        def in_copy(k):
            t, c0, ngr = items[k]
            b = wid * per_w + t
            return pltpu.make_async_copy(
                x_hbm.at[b, pl.ds(c0 * h, ngr * h), :],
                ibufs[k % 2].at[pl.ds(0, ngr * h)],
                isems[k % 2],
            )

        def ob_copy(k):
            t, c0, ngr = items[k]
            b = wid * per_w + t
            n = GR * hj if ngr == GR else TAILA
            return pltpu.make_async_copy(
                obufs[k % 2].at[pl.ds(0, n)],
                out_hbm.at[b, pl.ds(c0 * hj, n), :],
                osems[k % 2],
            )

        def tb_copy(k):
            t, c0, ngr = items[k]
            b = wid * per_w + t
            return pltpu.make_async_copy(
                tb,
                out_hbm.at[b, pl.ds(c0 * hj + TAILA, TAILN - TAILA), :],
                tsem,
            )

        all_cols = [lax.iota(jnp.int32, 16) + (c * 16) for c in range(NC16)]

        def copy_row(ib, src, dst_ref, dst_row):
            src_rows = jnp.full((16,), src, jnp.int32)
            dst_rows = jnp.full((16,), dst_row, jnp.int32)
            for cols in all_cols:
                v = plsc.load_gather(ib, [src_rows, cols])
                plsc.store_scatter(dst_ref, [dst_rows, cols], v)

        def repack(k):
            _, _, ngr = items[k]
            ib, ob = ibufs[k % 2], obufs[k % 2]

            for q in range(ngr):
                n_rows = hj
                if ngr != GR and q == ngr - 1:
                    n_rows = TAILA - q * hj

                def body(jj, _, q=q):
                    copy_row(ib, q * h + 1 + jj, ob, q * hj + jj)
                    return 0

                lax.fori_loop(0, n_rows, body, 0)
            if ngr != GR:
                for r in range(TAILA, TAILN):
                    q, jj = r // hj, r % hj
                    copy_row(ib, jnp.int32(q * h + 1 + jj),
                             tb, jnp.int32(r - TAILA))

        prev_out = {}
        last_tail = None
        in_copy(0).start()
        for k in range(n_items):
            slot = k % 2
            in_copy(k).wait()
            if k + 1 < n_items:
                in_copy(k + 1).start()
            if slot in prev_out:
                ob_copy(prev_out[slot]).wait()
            if is_tail(k) and last_tail is not None:
                tb_copy(last_tail).wait()
            repack(k)
            ob_copy(k).start()
            prev_out[slot] = k
            if is_tail(k):
                tb_copy(k).start()
                last_tail = k
        for slot in prev_out:
            ob_copy(prev_out[slot]).wait()
        if last_tail is not None:
            tb_copy(last_tail).wait()

    return copy_k(x)


# speedup vs baseline: 1.2959x; 1.2959x over previous
import functools

import jax
import jax.numpy as jnp
from jax import lax
from jax.experimental import pallas as pl
from jax.experimental.pallas import tpu as pltpu
from jax.experimental.pallas import tpu_sc as plsc


def kernel(x):
    B, S, C = x.shape
    h = int(round(S ** 0.5))
    hi, hj = h - 1, h - 2
    K = hi * hj

    info = plsc.get_sparse_core_info()
    nw = info.num_cores * info.num_subcores
    per_w = B // nw

    GR = 4
    chunks = []
    i0 = 0
    while i0 < hi:
        chunks.append((i0, min(GR, hi - i0)))
        i0 += GR
    items = [(t, c0, ngr) for t in range(per_w) for (c0, ngr) in chunks]
    n_items = len(items)
    NC16 = C // 16
    TAILN = (hi % GR) * hj
    TAILA = TAILN - (TAILN % 8)

    mesh = plsc.VectorSubcoreMesh(core_axis_name="c", subcore_axis_name="s")

    @functools.partial(
        pl.kernel,
        mesh=mesh,
        out_type=jax.ShapeDtypeStruct((B, K, C), jnp.float32),
        scratch_types=[
            pltpu.VMEM((GR * h, C), jnp.float32),
            pltpu.VMEM((GR * h, C), jnp.float32),
            pltpu.VMEM((GR * hj, C), jnp.float32),
            pltpu.VMEM((GR * hj, C), jnp.float32),
            pltpu.VMEM((TAILN - TAILA, C), jnp.float32),
            pltpu.SemaphoreType.DMA,
            pltpu.SemaphoreType.DMA,
            pltpu.SemaphoreType.DMA,
            pltpu.SemaphoreType.DMA,
            pltpu.SemaphoreType.DMA,
        ],
        compiler_params=pltpu.CompilerParams(
            needs_layout_passes=False,
            disable_bounds_checks=True,
            disable_semaphore_checks=True,
        ),
    )
    def copy_k(x_hbm, out_hbm, ib0, ib1, ob0, ob1, tb,
               is0, is1, os0, os1, tsem):
        wid = lax.axis_index("s") * info.num_cores + lax.axis_index("c")
        ibufs, obufs = (ib0, ib1), (ob0, ob1)
        isems, osems = (is0, is1), (os0, os1)

        def is_tail(k):
            return items[k][2] != GR

        def in_copy(k):
            t, c0, ngr = items[k]
            b = wid * per_w + t
            return pltpu.make_async_copy(
                x_hbm.at[b, pl.ds(c0 * h, ngr * h), :],
                ibufs[k % 2].at[pl.ds(0, ngr * h)],
                isems[k % 2],
            )

        def ob_copy(k):
            t, c0, ngr = items[k]
            b = wid * per_w + t
            n = GR * hj if ngr == GR else TAILA
            return pltpu.make_async_copy(
                obufs[k % 2].at[pl.ds(0, n)],
                out_hbm.at[b, pl.ds(c0 * hj, n), :],
                osems[k % 2],
            )

        def tb_copy(k):
            t, c0, ngr = items[k]
            b = wid * per_w + t
            return pltpu.make_async_copy(
                tb,
                out_hbm.at[b, pl.ds(c0 * hj + TAILA, TAILN - TAILA), :],
                tsem,
            )

        all_cols = [lax.iota(jnp.int32, 16) + (c * 16) for c in range(NC16)]

        def copy_row(ib, src, dst_ref, dst_row):
            src_rows = jnp.full((16,), src, jnp.int32)
            dst_rows = jnp.full((16,), dst_row, jnp.int32)
            for cols in all_cols:
                v = plsc.load_gather(ib, [src_rows, cols])
                plsc.store_scatter(dst_ref, [dst_rows, cols], v)

        def repack(k):
            _, _, ngr = items[k]
            ib, ob = ibufs[k % 2], obufs[k % 2]
            n_main = ngr * hj if ngr == GR else TAILA

            @plsc.parallel_loop(0, n_main, 1, unroll=2)
            def _(r):
                src = (r // hj) * h + (r % hj) + 1
                copy_row(ib, src, ob, r)

            if ngr != GR:
                for r in range(TAILA, TAILN):
                    q, jj = r // hj, r % hj
                    copy_row(ib, jnp.int32(q * h + 1 + jj),
                             tb, jnp.int32(r - TAILA))

        prev_out = {}
        last_tail = None
        in_copy(0).start()
        for k in range(n_items):
            slot = k % 2
            in_copy(k).wait()
            if k + 1 < n_items:
                in_copy(k + 1).start()
            if slot in prev_out:
                ob_copy(prev_out[slot]).wait()
            if is_tail(k) and last_tail is not None:
                tb_copy(last_tail).wait()
            repack(k)
            ob_copy(k).start()
            prev_out[slot] = k
            if is_tail(k):
                tb_copy(k).start()
                last_tail = k
        for slot in prev_out:
            ob_copy(prev_out[slot]).wait()
        if last_tail is not None:
            tb_copy(last_tail).wait()

    return copy_k(x)


# speedup vs baseline: 1.2992x; 1.0026x over previous
import functools

import jax
import jax.numpy as jnp
from jax import lax
from jax.experimental import pallas as pl
from jax.experimental.pallas import tpu as pltpu
from jax.experimental.pallas import tpu_sc as plsc


def kernel(x):
    B, S, C = x.shape
    h = int(round(S ** 0.5))
    hi, hj = h - 1, h - 2
    K = hi * hj

    info = plsc.get_sparse_core_info()
    nw = info.num_cores * info.num_subcores
    per_w = B // nw

    GR = 4
    chunks = []
    i0 = 0
    while i0 < hi:
        chunks.append((i0, min(GR, hi - i0)))
        i0 += GR
    items = [(t, c0, ngr) for t in range(per_w) for (c0, ngr) in chunks]
    n_items = len(items)
    NC16 = C // 16
    TAILN = (hi % GR) * hj
    TAILA = TAILN - (TAILN % 8)

    mesh = plsc.VectorSubcoreMesh(core_axis_name="c", subcore_axis_name="s")

    @functools.partial(
        pl.kernel,
        mesh=mesh,
        out_type=jax.ShapeDtypeStruct((B, K, C), jnp.float32),
        scratch_types=[
            pltpu.VMEM((GR * h, C), jnp.float32),
            pltpu.VMEM((GR * h, C), jnp.float32),
            pltpu.VMEM((GR * hj, C), jnp.float32),
            pltpu.VMEM((GR * hj, C), jnp.float32),
            pltpu.VMEM((TAILN - TAILA, C), jnp.float32),
            pltpu.SemaphoreType.DMA,
            pltpu.SemaphoreType.DMA,
            pltpu.SemaphoreType.DMA,
            pltpu.SemaphoreType.DMA,
            pltpu.SemaphoreType.DMA,
        ],
        compiler_params=pltpu.CompilerParams(
            needs_layout_passes=False,
            disable_bounds_checks=True,
            disable_semaphore_checks=True,
        ),
    )
    def copy_k(x_hbm, out_hbm, ib0, ib1, ob0, ob1, tb,
               is0, is1, os0, os1, tsem):
        wid = lax.axis_index("s") * info.num_cores + lax.axis_index("c")
        ibufs, obufs = (ib0, ib1), (ob0, ob1)
        isems, osems = (is0, is1), (os0, os1)

        def is_tail(k):
            return items[k][2] != GR

        def in_copy(k):
            t, c0, ngr = items[k]
            b = wid * per_w + t
            return pltpu.make_async_copy(
                x_hbm.at[b, pl.ds(c0 * h, ngr * h), :],
                ibufs[k % 2].at[pl.ds(0, ngr * h)],
                isems[k % 2],
            )

        def ob_copy(k):
            t, c0, ngr = items[k]
            b = wid * per_w + t
            n = GR * hj if ngr == GR else TAILA
            return pltpu.make_async_copy(
                obufs[k % 2].at[pl.ds(0, n)],
                out_hbm.at[b, pl.ds(c0 * hj, n), :],
                osems[k % 2],
            )

        def tb_copy(k):
            t, c0, ngr = items[k]
            b = wid * per_w + t
            return pltpu.make_async_copy(
                tb,
                out_hbm.at[b, pl.ds(c0 * hj + TAILA, TAILN - TAILA), :],
                tsem,
            )

        all_cols = [lax.iota(jnp.int32, 16) + (c * 16) for c in range(NC16)]

        def copy_row(ib, src, dst_ref, dst_row):
            src_rows = jnp.full((16,), src, jnp.int32)
            dst_rows = jnp.full((16,), dst_row, jnp.int32)
            for cols in all_cols:
                v = plsc.load_gather(ib, [src_rows, cols])
                plsc.store_scatter(dst_ref, [dst_rows, cols], v)

        def repack(k):
            _, _, ngr = items[k]
            ib, ob = ibufs[k % 2], obufs[k % 2]
            n_main = ngr * hj if ngr == GR else TAILA

            @plsc.parallel_loop(0, n_main, 1, unroll=3)
            def _(r):
                src = (r // hj) * h + (r % hj) + 1
                copy_row(ib, src, ob, r)

            if ngr != GR:
                for r in range(TAILA, TAILN):
                    q, jj = r // hj, r % hj
                    copy_row(ib, jnp.int32(q * h + 1 + jj),
                             tb, jnp.int32(r - TAILA))

        prev_out = {}
        last_tail = None
        in_copy(0).start()
        for k in range(n_items):
            slot = k % 2
            in_copy(k).wait()
            if k + 1 < n_items:
                in_copy(k + 1).start()
            if slot in prev_out:
                ob_copy(prev_out[slot]).wait()
            if is_tail(k) and last_tail is not None:
                tb_copy(last_tail).wait()
            repack(k)
            ob_copy(k).start()
            prev_out[slot] = k
            if is_tail(k):
                tb_copy(k).start()
                last_tail = k
        for slot in prev_out:
            ob_copy(prev_out[slot]).wait()
        if last_tail is not None:
            tb_copy(last_tail).wait()

    return copy_k(x)
